# baseline (speedup 1.0000x reference)
import jax
import jax.numpy as jnp
from jax.experimental import pallas as pl
import jax.experimental.pallas.tpu as pltpu

BATCH = 2048
DIM = 2048
NUM_EXPERTS = 8
ATTN_GAIN_MEAN = 1.0 + 32.0 / 2048.0
KT = 512
JT = 1024
NJ = DIM // JT
NK = DIM // KT
NSTEPS = NJ * NK
NSLOTS = 2
PREP_CHUNK = 256


def _prep_kernel(x_ref, gw_ref, gb_ref, xbf_ref, w_ref):
    c = pl.program_id(0)

    @pl.when(c == 0)
    def _gate():
        logits = gw_ref[0:1, :] + ATTN_GAIN_MEAN * gw_ref[1:2, :] + gb_ref[...]
        m = jnp.max(logits, axis=1, keepdims=True)
        e = jnp.exp(logits - m)
        w_ref[...] = e / jnp.sum(e, axis=1, keepdims=True)

    x = x_ref[...]
    mu = jnp.mean(x, axis=1, keepdims=True)
    xc = x - mu
    std = jnp.sqrt(jnp.mean(xc * xc, axis=1, keepdims=True)) + 1e-6
    xbf_ref[...] = (xc / std).astype(jnp.bfloat16)


def _ex_copy(ex_hbm, ebuf_ref, sem, slot, k, j):
    return pltpu.make_async_copy(
        ex_hbm.at[:, pl.ds(k * KT, KT), pl.ds(j * JT, JT)],
        ebuf_ref.at[slot],
        sem.at[slot],
    )


def _main_kernel(xk_ref, w_ref, ex_hbm, out_ref, ebuf_ref, sem):
    s = pl.program_id(0)
    j = s // NK
    k = jax.lax.rem(s, NK)
    slot = jax.lax.rem(s, NSLOTS)

    @pl.when(s == 0)
    def _first_copy():
        _ex_copy(ex_hbm, ebuf_ref, sem, 0, 0, 0).start()

    @pl.when(s + 1 < NSTEPS)
    def _prefetch():
        sn = s + 1
        _ex_copy(ex_hbm, ebuf_ref, sem, 1 - slot,
                 jax.lax.rem(sn, NK), sn // NK).start()

    _ex_copy(ex_hbm, ebuf_ref, sem, slot, k, j).wait()

    ex = ebuf_ref[slot]
    comb = w_ref[0, 0] * ex[0]
    for e_idx in range(1, NUM_EXPERTS):
        comb = comb + w_ref[0, e_idx] * ex[e_idx]
    comb_bf = comb.astype(jnp.bfloat16)

    xk = xk_ref[...]
    quart = JT // 4
    for h in range(4):
        acc = jnp.dot(xk, comb_bf[:, h * quart:(h + 1) * quart],
                      preferred_element_type=jnp.float32)

        @pl.when(k == 0)
        def _fst():
            out_ref[:, pl.ds(h * quart, quart)] = acc

        @pl.when(k > 0)
        def _rst():
            out_ref[:, pl.ds(h * quart, quart)] += acc


@jax.jit
def kernel(query_embedding, gate_W, gate_b, experts):
    gate_b2 = gate_b.reshape(1, NUM_EXPERTS)
    xbf, w = pl.pallas_call(
        _prep_kernel,
        grid=(BATCH // PREP_CHUNK,),
        in_specs=[
            pl.BlockSpec((PREP_CHUNK, DIM), lambda c: (c, 0)),
            pl.BlockSpec((12, NUM_EXPERTS), lambda c: (0, 0)),
            pl.BlockSpec((1, NUM_EXPERTS), lambda c: (0, 0)),
        ],
        out_specs=[
            pl.BlockSpec((PREP_CHUNK, DIM), lambda c: (c, 0)),
            pl.BlockSpec((1, NUM_EXPERTS), lambda c: (0, 0)),
        ],
        out_shape=[
            jax.ShapeDtypeStruct((BATCH, DIM), jnp.bfloat16),
            jax.ShapeDtypeStruct((1, NUM_EXPERTS), jnp.float32),
        ],
    )(query_embedding, gate_W, gate_b2)

    return pl.pallas_call(
        _main_kernel,
        grid=(NSTEPS,),
        in_specs=[
            pl.BlockSpec((BATCH, KT), lambda s: (0, s % NK)),
            pl.BlockSpec((1, NUM_EXPERTS), lambda s: (0, 0)),
            pl.BlockSpec(memory_space=pl.ANY),
        ],
        out_specs=pl.BlockSpec((BATCH, JT), lambda s: (0, s // NK)),
        out_shape=jax.ShapeDtypeStruct((BATCH, DIM), jnp.float32),
        scratch_shapes=[
            pltpu.VMEM((NSLOTS, NUM_EXPERTS, KT, JT), jnp.float32),
            pltpu.SemaphoreType.DMA((NSLOTS,)),
        ],
    )(xbf, w, experts)


# speedup vs baseline: 1.3738x; 1.3738x over previous
import jax
import jax.numpy as jnp
from jax.experimental import pallas as pl
import jax.experimental.pallas.tpu as pltpu

BATCH = 2048
DIM = 2048
NUM_EXPERTS = 8
ATTN_GAIN_MEAN = 1.0 + 32.0 / 2048.0
KT = 1024
JT = 512
NJ = DIM // JT
NK = DIM // KT
NSTEPS = NJ * NK
PREP_CHUNK = 256


def _prep_kernel(x_ref, gw_ref, gb_ref, xbf_ref, w_ref):
    c = pl.program_id(0)

    @pl.when(c == 0)
    def _gate():
        logits = gw_ref[0:1, :] + ATTN_GAIN_MEAN * gw_ref[1:2, :] + gb_ref[...]
        m = jnp.max(logits, axis=1, keepdims=True)
        e = jnp.exp(logits - m)
        w_ref[...] = e / jnp.sum(e, axis=1, keepdims=True)

    x = x_ref[...]
    mu = jnp.mean(x, axis=1, keepdims=True)
    xc = x - mu
    std = jnp.sqrt(jnp.mean(xc * xc, axis=1, keepdims=True)) + 1e-6
    xbf_ref[...] = (xc / std).astype(jnp.bfloat16)


def _ex_copy(ex_hbm, ebuf_ref, sem, slot, k, j):
    return pltpu.make_async_copy(
        ex_hbm.at[:, pl.ds(k * KT, KT), pl.ds(j * JT, JT)],
        ebuf_ref.at[slot],
        sem.at[slot],
    )


def _main_kernel(xbf_ref, w_ref, ex_hbm, out_ref, ebuf_ref, sem):
    s = pl.program_id(0)
    j = s // NK
    k = jax.lax.rem(s, NK)
    slot = jax.lax.rem(s, 2)

    @pl.when(s == 0)
    def _first_copy():
        _ex_copy(ex_hbm, ebuf_ref, sem, 0, 0, 0).start()

    @pl.when(s + 1 < NSTEPS)
    def _prefetch():
        sn = s + 1
        _ex_copy(ex_hbm, ebuf_ref, sem, 1 - slot,
                 jax.lax.rem(sn, NK), sn // NK).start()

    _ex_copy(ex_hbm, ebuf_ref, sem, slot, k, j).wait()

    ex = ebuf_ref[slot]
    comb = w_ref[0, 0] * ex[0]
    for e_idx in range(1, NUM_EXPERTS):
        comb = comb + w_ref[0, e_idx] * ex[e_idx]

    xk = xbf_ref[:, pl.ds(k * KT, KT)]
    acc = jnp.dot(xk, comb.astype(jnp.bfloat16),
                  preferred_element_type=jnp.float32)

    @pl.when(k == 0)
    def _fst():
        out_ref[...] = acc

    @pl.when(k > 0)
    def _rst():
        out_ref[...] += acc


@jax.jit
def kernel(query_embedding, gate_W, gate_b, experts):
    gate_b2 = gate_b.reshape(1, NUM_EXPERTS)
    xbf, w = pl.pallas_call(
        _prep_kernel,
        grid=(BATCH // PREP_CHUNK,),
        in_specs=[
            pl.BlockSpec((PREP_CHUNK, DIM), lambda c: (c, 0)),
            pl.BlockSpec((12, NUM_EXPERTS), lambda c: (0, 0)),
            pl.BlockSpec((1, NUM_EXPERTS), lambda c: (0, 0)),
        ],
        out_specs=[
            pl.BlockSpec((PREP_CHUNK, DIM), lambda c: (c, 0)),
            pl.BlockSpec((1, NUM_EXPERTS), lambda c: (0, 0)),
        ],
        out_shape=[
            jax.ShapeDtypeStruct((BATCH, DIM), jnp.bfloat16),
            jax.ShapeDtypeStruct((1, NUM_EXPERTS), jnp.float32),
        ],
    )(query_embedding, gate_W, gate_b2)

    return pl.pallas_call(
        _main_kernel,
        grid=(NSTEPS,),
        in_specs=[
            pl.BlockSpec((BATCH, DIM), lambda s: (0, 0)),
            pl.BlockSpec((1, NUM_EXPERTS), lambda s: (0, 0)),
            pl.BlockSpec(memory_space=pl.ANY),
        ],
        out_specs=pl.BlockSpec((BATCH, JT), lambda s: (0, s // NK)),
        out_shape=jax.ShapeDtypeStruct((BATCH, DIM), jnp.float32),
        scratch_shapes=[
            pltpu.VMEM((2, NUM_EXPERTS, KT, JT), jnp.float32),
            pltpu.SemaphoreType.DMA((2,)),
        ],
    )(xbf, w, experts)
